# initial kernel scaffold (unmeasured)
import jax
import jax.numpy as jnp
from jax import lax
from jax.experimental import pallas as pl
from jax.experimental.pallas import tpu as pltpu

N_DEV = 4


def _gelu(y):
    c = 0.7978845608028654
    return 0.5 * y * (1.0 + jnp.tanh(c * (y + 0.044715 * y * y * y)))


def kernel(x, w_mat):
    m_tot, k_loc = x.shape
    _, n_tot = w_mat.shape
    m_per = m_tot // N_DEV
    n_half = n_tot // 2

    def body(x_ref, w_ref, out_ref,
             acc_r, acc_l, recv_r, recv_l,
             send_sem_r, send_sem_l, recv_sems_r, recv_sems_l):
        me = lax.axis_index("i")
        right = lax.rem(me + 1, N_DEV)
        left = lax.rem(me + N_DEV - 1, N_DEV)

        barrier = pltpu.get_barrier_semaphore()
        for nbr in (left, right):
            pl.semaphore_signal(
                barrier, inc=1,
                device_id=(nbr,), device_id_type=pl.DeviceIdType.MESH,
            )
        pl.semaphore_wait(barrier, 2)

        def partial(c, col_lo):
            xs = x_ref[pl.ds(c * m_per, m_per), :]
            ws = w_ref[:, pl.ds(col_lo, n_half)]
            return jnp.dot(xs, ws, preferred_element_type=jnp.float32)

        for h in range(N_DEV - 1):
            c_r = lax.rem(me - 1 - h + 2 * N_DEV, N_DEV)
            c_l = lax.rem(me + 1 + h, N_DEV)
            p_r = partial(c_r, 0)
            p_l = partial(c_l, n_half)
            if h > 0:
                p_r = p_r + recv_r[h - 1].astype(jnp.float32)
                p_l = p_l + recv_l[h - 1].astype(jnp.float32)
            acc_r[...] = p_r.astype(jnp.bfloat16)
            acc_l[...] = p_l.astype(jnp.bfloat16)
            rdma_r = pltpu.make_async_remote_copy(
                src_ref=acc_r, dst_ref=recv_r.at[h],
                send_sem=send_sem_r, recv_sem=recv_sems_r.at[h],
                device_id=(right,), device_id_type=pl.DeviceIdType.MESH,
            )
            rdma_l = pltpu.make_async_remote_copy(
                src_ref=acc_l, dst_ref=recv_l.at[h],
                send_sem=send_sem_l, recv_sem=recv_sems_l.at[h],
                device_id=(left,), device_id_type=pl.DeviceIdType.MESH,
            )
            rdma_r.start()
            rdma_l.start()
            rdma_r.wait()
            rdma_l.wait()

        p_r = partial(me, 0) + recv_r[N_DEV - 2].astype(jnp.float32)
        p_l = partial(me, n_half) + recv_l[N_DEV - 2].astype(jnp.float32)
        out_ref[:, :n_half] = _gelu(p_r)
        out_ref[:, n_half:] = _gelu(p_l)

    return pl.pallas_call(
        body,
        out_shape=jax.ShapeDtypeStruct((m_per, n_tot), jnp.float32),
        in_specs=[
            pl.BlockSpec(memory_space=pltpu.VMEM),
            pl.BlockSpec(memory_space=pltpu.VMEM),
        ],
        out_specs=pl.BlockSpec(memory_space=pltpu.VMEM),
        scratch_shapes=[
            pltpu.VMEM((m_per, n_half), jnp.bfloat16),
            pltpu.VMEM((m_per, n_half), jnp.bfloat16),
            pltpu.VMEM((N_DEV - 1, m_per, n_half), jnp.bfloat16),
            pltpu.VMEM((N_DEV - 1, m_per, n_half), jnp.bfloat16),
            pltpu.SemaphoreType.DMA,
            pltpu.SemaphoreType.DMA,
            pltpu.SemaphoreType.DMA((N_DEV - 1,)),
            pltpu.SemaphoreType.DMA((N_DEV - 1,)),
        ],
        compiler_params=pltpu.CompilerParams(collective_id=0),
    )(x, w_mat)


# baseline (device time: 345710 ns/iter reference)
import jax
import jax.numpy as jnp
from jax import lax
from jax.experimental import pallas as pl
from jax.experimental.pallas import tpu as pltpu

N_DEV = 4
T = 512


def _gelu(y):
    c = 0.7978845608028654
    return 0.5 * y * (1.0 + jnp.tanh(c * (y + 0.044715 * y * y * y)))


def kernel(x, w_mat):
    m_tot, k_loc = x.shape
    _, n_tot = w_mat.shape
    m_per = m_tot // N_DEV
    n_half = n_tot // 2
    nt = n_half // T

    xb = x.astype(jnp.bfloat16)
    wb = w_mat.astype(jnp.bfloat16)

    def body(x_ref, w_ref, out_ref,
             a_r, b_r, a_l, b_l,
             send_sems_r, send_sems_l, recv_sems_r, recv_sems_l,
             credit_r, credit_l, out_sems_r, out_sems_l):
        me = lax.axis_index("i")
        right = lax.rem(me + 1, N_DEV)
        left = lax.rem(me + N_DEV - 1, N_DEV)

        barrier = pltpu.get_barrier_semaphore()
        for nbr in (left, right):
            pl.semaphore_signal(
                barrier, inc=1,
                device_id=(nbr,), device_id_type=pl.DeviceIdType.MESH,
            )
        pl.semaphore_wait(barrier, 2)

        def mm(c, col0, t):
            xs = x_ref[pl.ds(c * m_per, m_per), :]
            ws = w_ref[:, pl.ds(col0 + t * T, T)]
            return jnp.dot(xs, ws, preferred_element_type=jnp.float32)

        rd_r = [None] * ((N_DEV - 1) * nt)
        rd_l = [None] * ((N_DEV - 1) * nt)

        def hop_stripe(h, t, a, b, send_sems, recv_sems, credit, rd,
                       col0, dst, upstream, rightward):
            k = h * nt + t
            slot = k % 2
            if k >= 2:
                rd[k - 2].wait_send()
            if rightward:
                c = lax.rem(me - 1 - h + 2 * N_DEV, N_DEV)
            else:
                c = lax.rem(me + 1 + h, N_DEV)
            if h == 0:
                val = mm(c, col0, t)
            else:
                rd[k - nt].wait_recv()
                val = mm(c, col0, t) + b[t].astype(jnp.float32)
                pl.semaphore_signal(
                    credit, inc=1,
                    device_id=(upstream,),
                    device_id_type=pl.DeviceIdType.MESH,
                )
            a[slot] = val.astype(jnp.bfloat16)
            if h >= 1:
                pl.semaphore_wait(credit, 1)
            rd[k] = pltpu.make_async_remote_copy(
                src_ref=a.at[slot], dst_ref=b.at[t],
                send_sem=send_sems.at[slot], recv_sem=recv_sems.at[t],
                device_id=(dst,), device_id_type=pl.DeviceIdType.MESH,
            )
            rd[k].start()

        for h in range(N_DEV - 1):
            for t in range(nt):
                hop_stripe(h, t, a_r, b_r, send_sems_r, recv_sems_r,
                           credit_r, rd_r, 0, right, left, True)
                hop_stripe(h, t, a_l, b_l, send_sems_l, recv_sems_l,
                           credit_l, rd_l, n_half, left, right, False)

        cp_r = [None] * nt
        cp_l = [None] * nt
        last = (N_DEV - 2) * nt

        def final_stripe(t, a, b, rd, cp, col0, out_sems):
            slot = t % 2
            if t < 2:
                rd[last + nt - 2 + t].wait_send()
            else:
                cp[t - 2].wait()
            rd[last + t].wait_recv()
            val = mm(me, col0, t) + b[t].astype(jnp.float32)
            a[slot] = _gelu(val).astype(jnp.bfloat16)
            cp[t] = pltpu.make_async_copy(
                a.at[slot],
                out_ref.at[:, pl.ds(col0 + t * T, T)],
                out_sems.at[slot],
            )
            cp[t].start()

        for t in range(nt):
            final_stripe(t, a_r, b_r, rd_r, cp_r, 0, out_sems_r)
            final_stripe(t, a_l, b_l, rd_l, cp_l, n_half, out_sems_l)
        for cp in (cp_r[nt - 2], cp_r[nt - 1], cp_l[nt - 2], cp_l[nt - 1]):
            cp.wait()

    return pl.pallas_call(
        body,
        out_shape=jax.ShapeDtypeStruct((m_per, n_tot), jnp.bfloat16),
        in_specs=[
            pl.BlockSpec(memory_space=pltpu.MemorySpace.VMEM),
            pl.BlockSpec(memory_space=pltpu.MemorySpace.VMEM),
        ],
        out_specs=pl.BlockSpec(memory_space=pl.ANY),
        scratch_shapes=[
            pltpu.VMEM((2, m_per, T), jnp.bfloat16),
            pltpu.VMEM((nt, m_per, T), jnp.bfloat16),
            pltpu.VMEM((2, m_per, T), jnp.bfloat16),
            pltpu.VMEM((nt, m_per, T), jnp.bfloat16),
            pltpu.SemaphoreType.DMA((2,)),
            pltpu.SemaphoreType.DMA((2,)),
            pltpu.SemaphoreType.DMA((nt,)),
            pltpu.SemaphoreType.DMA((nt,)),
            pltpu.SemaphoreType.REGULAR,
            pltpu.SemaphoreType.REGULAR,
            pltpu.SemaphoreType.DMA((2,)),
            pltpu.SemaphoreType.DMA((2,)),
        ],
        compiler_params=pltpu.CompilerParams(
            collective_id=0,
            vmem_limit_bytes=39 * 1024 * 1024,
        ),
    )(xb, wb)


# device time: 332836 ns/iter; 1.0387x vs baseline; 1.0387x over previous
import jax
import jax.numpy as jnp
from jax import lax
from jax.experimental import pallas as pl
from jax.experimental.pallas import tpu as pltpu

N_DEV = 4
T = 512
SUB = 256
LB = 512


def _gelu(y):
    c = 0.7978845608028654
    return 0.5 * y * (1.0 + jnp.tanh(c * (y + 0.044715 * y * y * y)))


def kernel(x, w_mat):
    m_tot, k_loc = x.shape
    _, n_tot = w_mat.shape
    m_per = m_tot // N_DEV
    n_half = n_tot // 2
    nt = n_half // T

    def body(x_hbm, w_hbm, out_ref,
             xb, wb, stage,
             a_r, b_r, a_l, b_l,
             send_sems_r, send_sems_l, recv_sems_r, recv_sems_l,
             credit_r, credit_l, out_sems_r, out_sems_l, load_sem):
        me = lax.axis_index("i")
        right = lax.rem(me + 1, N_DEV)
        left = lax.rem(me + N_DEV - 1, N_DEV)

        barrier = pltpu.get_barrier_semaphore()
        for nbr in (left, right):
            pl.semaphore_signal(
                barrier, inc=1,
                device_id=(nbr,), device_id_type=pl.DeviceIdType.MESH,
            )

        def load_block(src, dst_rows, dst_cols, dst):
            cp = pltpu.make_async_copy(src, stage, load_sem)
            cp.start()
            cp.wait()
            dst[dst_rows, dst_cols] = stage[...].astype(jnp.bfloat16)

        def load_w_stripe(t):
            for col0 in (t * T, n_half + t * T):
                for r in range(k_loc // LB):
                    rows = pl.ds(r * LB, LB)
                    cols = pl.ds(col0, LB)
                    load_block(w_hbm.at[rows, cols], rows, cols, wb)

        def load_x_chunk(c):
            for r in range(m_per // LB):
                rows = pl.ds(c * m_per + r * LB, LB)
                for kk in range(k_loc // LB):
                    cols = pl.ds(kk * LB, LB)
                    load_block(x_hbm.at[rows, cols], rows, cols, xb)

        def mm(c, col0, t, s):
            xs = xb[pl.ds(c * m_per, m_per), :]
            ws = wb[:, pl.ds(col0 + t * T + s * SUB, SUB)]
            return jnp.dot(xs, ws, preferred_element_type=jnp.float32)

        rd_r = [None] * ((N_DEV - 1) * nt)
        rd_l = [None] * ((N_DEV - 1) * nt)

        def hop_stripe(h, t, a, b, send_sems, recv_sems, credit, rd,
                       col0, dst, upstream, rightward):
            k = h * nt + t
            slot = k % 2
            if k >= 2:
                rd[k - 2].wait_send()
            if rightward:
                c = lax.rem(me - 1 - h + 2 * N_DEV, N_DEV)
            else:
                c = lax.rem(me + 1 + h, N_DEV)
            if h > 0:
                rd[k - nt].wait_recv()
            for s in range(T // SUB):
                cs = pl.ds(s * SUB, SUB)
                val = mm(c, col0, t, s)
                if h > 0:
                    val = val + b[t, :, cs].astype(jnp.float32)
                a[slot, :, cs] = val.astype(jnp.bfloat16)
            if h > 0:
                pl.semaphore_signal(
                    credit, inc=1,
                    device_id=(upstream,),
                    device_id_type=pl.DeviceIdType.MESH,
                )
                pl.semaphore_wait(credit, 1)
            rd[k] = pltpu.make_async_remote_copy(
                src_ref=a.at[slot], dst_ref=b.at[t],
                send_sem=send_sems.at[slot], recv_sem=recv_sems.at[t],
                device_id=(dst,), device_id_type=pl.DeviceIdType.MESH,
            )
            rd[k].start()

        def hop_r(h, t):
            hop_stripe(h, t, a_r, b_r, send_sems_r, recv_sems_r,
                       credit_r, rd_r, 0, right, left, True)

        def hop_l(h, t):
            hop_stripe(h, t, a_l, b_l, send_sems_l, recv_sems_l,
                       credit_l, rd_l, n_half, left, right, False)

        load_w_stripe(0)
        load_x_chunk(lax.rem(me - 1 + N_DEV, N_DEV))
        pl.semaphore_wait(barrier, 2)
        hop_r(0, 0)
        load_x_chunk(lax.rem(me + 1, N_DEV))
        hop_l(0, 0)
        for t in range(1, nt):
            load_w_stripe(t)
            hop_r(0, t)
            if t == 1:
                load_x_chunk(lax.rem(me + 2, N_DEV))
            if t == 2:
                load_x_chunk(me)
            hop_l(0, t)

        for h in range(1, N_DEV - 1):
            for t in range(nt):
                hop_r(h, t)
                hop_l(h, t)

        cp_r = [None] * nt
        cp_l = [None] * nt
        last = (N_DEV - 2) * nt

        def final_stripe(t, a, b, rd, cp, col0, out_sems):
            slot = t % 2
            if t < 2:
                rd[last + nt - 2 + t].wait_send()
            else:
                cp[t - 2].wait()
            rd[last + t].wait_recv()
            for s in range(T // SUB):
                cs = pl.ds(s * SUB, SUB)
                val = mm(me, col0, t, s) + b[t, :, cs].astype(jnp.float32)
                a[slot, :, cs] = _gelu(val).astype(jnp.bfloat16)
            cp[t] = pltpu.make_async_copy(
                a.at[slot],
                out_ref.at[:, pl.ds(col0 + t * T, T)],
                out_sems.at[slot],
            )
            cp[t].start()

        for t in range(nt):
            final_stripe(t, a_r, b_r, rd_r, cp_r, 0, out_sems_r)
            final_stripe(t, a_l, b_l, rd_l, cp_l, n_half, out_sems_l)
        for cp in (cp_r[nt - 2], cp_r[nt - 1], cp_l[nt - 2], cp_l[nt - 1]):
            cp.wait()

    return pl.pallas_call(
        body,
        out_shape=jax.ShapeDtypeStruct((m_per, n_tot), jnp.bfloat16),
        in_specs=[
            pl.BlockSpec(memory_space=pl.ANY),
            pl.BlockSpec(memory_space=pl.ANY),
        ],
        out_specs=pl.BlockSpec(memory_space=pl.ANY),
        scratch_shapes=[
            pltpu.VMEM((m_tot, k_loc), jnp.bfloat16),
            pltpu.VMEM((k_loc, n_tot), jnp.bfloat16),
            pltpu.VMEM((LB, LB), jnp.float32),
            pltpu.VMEM((2, m_per, T), jnp.bfloat16),
            pltpu.VMEM((nt, m_per, T), jnp.bfloat16),
            pltpu.VMEM((2, m_per, T), jnp.bfloat16),
            pltpu.VMEM((nt, m_per, T), jnp.bfloat16),
            pltpu.SemaphoreType.DMA((2,)),
            pltpu.SemaphoreType.DMA((2,)),
            pltpu.SemaphoreType.DMA((nt,)),
            pltpu.SemaphoreType.DMA((nt,)),
            pltpu.SemaphoreType.REGULAR,
            pltpu.SemaphoreType.REGULAR,
            pltpu.SemaphoreType.DMA((2,)),
            pltpu.SemaphoreType.DMA((2,)),
            pltpu.SemaphoreType.DMA,
        ],
        compiler_params=pltpu.CompilerParams(
            collective_id=0,
            vmem_limit_bytes=62 * 1024 * 1024,
        ),
    )(x, w_mat)


# device time: 321257 ns/iter; 1.0761x vs baseline; 1.0360x over previous
import jax
import jax.numpy as jnp
from jax import lax
from jax.experimental import pallas as pl
from jax.experimental.pallas import tpu as pltpu

N_DEV = 4
T = 512
SUB = 256
LB = 512


def _gelu(y):
    c = 0.7978845608028654
    return 0.5 * y * (1.0 + jnp.tanh(c * (y + 0.044715 * y * y * y)))


def kernel(x, w_mat):
    m_tot, k_loc = x.shape
    _, n_tot = w_mat.shape
    m_per = m_tot // N_DEV
    n_half = n_tot // 2
    nt = n_half // T

    def body(x_hbm, w_hbm, out_ref,
             xb, wb, stage,
             a_r, b_r, a_l, b_l,
             send_sems_r, send_sems_l, recv_sems_r, recv_sems_l,
             credit_r, credit_l, out_sems_r, out_sems_l, load_sems):
        me = lax.axis_index("i")
        right = lax.rem(me + 1, N_DEV)
        left = lax.rem(me + N_DEV - 1, N_DEV)

        barrier = pltpu.get_barrier_semaphore()
        for nbr in (left, right):
            pl.semaphore_signal(
                barrier, inc=1,
                device_id=(nbr,), device_id_type=pl.DeviceIdType.MESH,
            )

        pend = []

        def load_block(src, dst):
            s = len(pend) % 2
            if len(pend) >= 2:
                c0, d0, s0 = pend[-2]
                c0.wait()
                d0[...] = stage[s0].astype(jnp.bfloat16)
            cp = pltpu.make_async_copy(src, stage.at[s], load_sems.at[s])
            cp.start()
            pend.append((cp, dst, s))

        def drain_loads():
            for c0, d0, s0 in pend[-2:]:
                c0.wait()
                d0[...] = stage[s0].astype(jnp.bfloat16)
            pend.clear()

        def load_w_stripe(t):
            for col0 in (t * T, n_half + t * T):
                for r in range(k_loc // LB):
                    rows = pl.ds(r * LB, LB)
                    cols = pl.ds(col0, LB)
                    load_block(w_hbm.at[rows, cols], wb.at[rows, cols])

        def load_x_chunk(c, j):
            for r in range(m_per // LB):
                rows = pl.ds(c * m_per + r * LB, LB)
                for kk in range(k_loc // LB):
                    cols = pl.ds(kk * LB, LB)
                    load_block(x_hbm.at[rows, cols],
                               xb.at[j, pl.ds(r * LB, LB), cols])

        def mm(j, col0, t, s):
            xs = xb[j]
            ws = wb[:, pl.ds(col0 + t * T + s * SUB, SUB)]
            return jnp.dot(xs, ws, preferred_element_type=jnp.float32)

        rd_r = [None] * ((N_DEV - 1) * nt)
        rd_l = [None] * ((N_DEV - 1) * nt)

        def hop_stripe(h, t, a, b, send_sems, recv_sems, credit, rd,
                       col0, dst, upstream, rightward):
            k = h * nt + t
            slot = k % 2
            if k >= 2:
                rd[k - 2].wait_send()
            j = ([0, 2, 1] if rightward else [1, 2, 0])[h]
            if h > 0:
                rd[k - nt].wait_recv()
            for s in range(T // SUB):
                cs = pl.ds(s * SUB, SUB)
                val = mm(j, col0, t, s)
                if h > 0:
                    val = val + b[t, :, cs].astype(jnp.float32)
                a[slot, :, cs] = val.astype(jnp.bfloat16)
            if h > 0:
                pl.semaphore_signal(
                    credit, inc=1,
                    device_id=(upstream,),
                    device_id_type=pl.DeviceIdType.MESH,
                )
                pl.semaphore_wait(credit, 1)
            rd[k] = pltpu.make_async_remote_copy(
                src_ref=a.at[slot], dst_ref=b.at[t],
                send_sem=send_sems.at[slot], recv_sem=recv_sems.at[t],
                device_id=(dst,), device_id_type=pl.DeviceIdType.MESH,
            )
            rd[k].start()

        def hop_r(h, t):
            hop_stripe(h, t, a_r, b_r, send_sems_r, recv_sems_r,
                       credit_r, rd_r, 0, right, left, True)

        def hop_l(h, t):
            hop_stripe(h, t, a_l, b_l, send_sems_l, recv_sems_l,
                       credit_l, rd_l, n_half, left, right, False)

        load_w_stripe(0)
        load_x_chunk(lax.rem(me - 1 + N_DEV, N_DEV), 0)
        drain_loads()
        pl.semaphore_wait(barrier, 2)
        hop_r(0, 0)
        load_x_chunk(lax.rem(me + 1, N_DEV), 1)
        drain_loads()
        hop_l(0, 0)
        for t in range(1, nt):
            load_w_stripe(t)
            if t == 1:
                load_x_chunk(lax.rem(me + 2, N_DEV), 2)
            if t == 2:
                load_x_chunk(me, 3)
            drain_loads()
            hop_r(0, t)
            hop_l(0, t)

        for h in range(1, N_DEV - 1):
            for t in range(nt):
                hop_r(h, t)
                hop_l(h, t)

        cp_r = [None] * nt
        cp_l = [None] * nt
        last = (N_DEV - 2) * nt

        def final_stripe(t, a, b, rd, cp, col0, out_sems):
            slot = t % 2
            if t < 2:
                rd[last + nt - 2 + t].wait_send()
            else:
                cp[t - 2].wait()
            rd[last + t].wait_recv()
            for s in range(T // SUB):
                cs = pl.ds(s * SUB, SUB)
                val = mm(3, col0, t, s) + b[t, :, cs].astype(jnp.float32)
                a[slot, :, cs] = _gelu(val).astype(jnp.bfloat16)
            cp[t] = pltpu.make_async_copy(
                a.at[slot],
                out_ref.at[:, pl.ds(col0 + t * T, T)],
                out_sems.at[slot],
            )
            cp[t].start()

        for t in range(nt):
            final_stripe(t, a_r, b_r, rd_r, cp_r, 0, out_sems_r)
            final_stripe(t, a_l, b_l, rd_l, cp_l, n_half, out_sems_l)
        for cp in (cp_r[nt - 2], cp_r[nt - 1], cp_l[nt - 2], cp_l[nt - 1]):
            cp.wait()

    return pl.pallas_call(
        body,
        out_shape=jax.ShapeDtypeStruct((m_per, n_tot), jnp.bfloat16),
        in_specs=[
            pl.BlockSpec(memory_space=pl.ANY),
            pl.BlockSpec(memory_space=pl.ANY),
        ],
        out_specs=pl.BlockSpec(memory_space=pl.ANY),
        scratch_shapes=[
            pltpu.VMEM((N_DEV, m_per, k_loc), jnp.bfloat16),
            pltpu.VMEM((k_loc, n_tot), jnp.bfloat16),
            pltpu.VMEM((2, LB, LB), jnp.float32),
            pltpu.VMEM((2, m_per, T), jnp.bfloat16),
            pltpu.VMEM((nt, m_per, T), jnp.bfloat16),
            pltpu.VMEM((2, m_per, T), jnp.bfloat16),
            pltpu.VMEM((nt, m_per, T), jnp.bfloat16),
            pltpu.SemaphoreType.DMA((2,)),
            pltpu.SemaphoreType.DMA((2,)),
            pltpu.SemaphoreType.DMA((nt,)),
            pltpu.SemaphoreType.DMA((nt,)),
            pltpu.SemaphoreType.REGULAR,
            pltpu.SemaphoreType.REGULAR,
            pltpu.SemaphoreType.DMA((2,)),
            pltpu.SemaphoreType.DMA((2,)),
            pltpu.SemaphoreType.DMA((2,)),
        ],
        compiler_params=pltpu.CompilerParams(
            collective_id=0,
            vmem_limit_bytes=62 * 1024 * 1024,
        ),
    )(x, w_mat)
